# Initial kernel scaffold; baseline (speedup 1.0000x reference)
import jax
import jax.numpy as jnp
from jax.experimental import pallas as pl


def kernel(x, target):
    raise NotImplementedError("write your pallas kernel here")


# speedup vs baseline: 1.8127x; 1.8127x over previous
import math

import jax
import jax.numpy as jnp
from jax import lax
from jax.experimental import pallas as pl
from jax.experimental.pallas import tpu as pltpu

_SMOOTHING = 0.1
_CONFIDENCE = 1.0 - _SMOOTHING
_N = 1024
_V = 100000
_EPS = _SMOOTHING / (_V - 1)
_CONST = _N * ((_V - 1) * _EPS * math.log(_EPS) + _CONFIDENCE * math.log(_CONFIDENCE))

_RB = 32
_NB = _N // _RB


def _body(tgt_ref, x_ref, out_ref):
    b = pl.program_id(0)

    @pl.when(b == 0)
    def _init():
        out_ref[...] = jnp.full((1, 1), _CONST, jnp.float32)

    xb = x_ref[...]
    tgt = tgt_ref[0, 0, :]
    col = lax.broadcasted_iota(jnp.int32, (_RB, _V), 1)
    is_target = col == tgt[:, None]
    w = jnp.where(is_target, jnp.float32(_CONFIDENCE), jnp.float32(_EPS))
    out_ref[...] -= jnp.sum(xb * w).reshape(1, 1)


def kernel(x, target):
    tgt = target.astype(jnp.int32).reshape(_NB, 1, _RB)
    out = pl.pallas_call(
        _body,
        grid=(_NB,),
        in_specs=[
            pl.BlockSpec((1, 1, _RB), lambda b: (b, 0, 0)),
            pl.BlockSpec((_RB, _V), lambda b: (b, 0)),
        ],
        out_specs=pl.BlockSpec((1, 1), lambda b: (0, 0)),
        out_shape=jax.ShapeDtypeStruct((1, 1), jnp.float32),
        compiler_params=pltpu.CompilerParams(
            dimension_semantics=("arbitrary",),
        ),
    )(tgt, x)
    return out[0, 0]
